# baseline (speedup 1.0000x reference)
import functools

import jax
import jax.numpy as jnp
from jax import lax
from jax.experimental import pallas as pl
from jax.experimental.pallas import tpu as pltpu
from jax.experimental.pallas import tpu_sc as plsc

N_NODES = 10000
D = 128
E = 320000
NC, NS = 2, 16
NW = NC * NS
CHUNK = 128
N_CHUNKS = E // CHUNK
BASE_CPW = N_CHUNKS // NW
EXTRA = N_CHUNKS % NW
N_ACC = 10240
ROWS_PER_TILE = N_ACC // NS
CW = 16


def _sc_aggregate_body(with_counts, table, src, dst, out, out_cnt,
                       ibuf_s, ibuf_d, rows_0, rows_1, acc, cnt, ones_buf,
                       sem_is0, sem_is1, sem_id0, sem_id1,
                       sem_g0a, sem_g0b, sem_g0c, sem_g0d,
                       sem_g1a, sem_g1b, sem_g1c, sem_g1d,
                       sem_m0, sem_m1, sem_c0, sem_c1, sem_z):
    cid = lax.axis_index("c")
    sid = lax.axis_index("s")
    wid = sid * NC + cid
    nreal = BASE_CPW + jnp.where(wid < EXTRA, 1, 0)
    start = BASE_CPW * wid + jnp.minimum(wid, EXTRA)

    rows = (rows_0, rows_1)
    sem_is = (sem_is0, sem_is1)
    sem_id = (sem_id0, sem_id1)
    sem_g = ((sem_g0a, sem_g0b, sem_g0c, sem_g0d),
             (sem_g1a, sem_g1b, sem_g1c, sem_g1d))
    sem_m = (sem_m0, sem_m1)
    sem_c = (sem_c0, sem_c1)

    def _zero_rows(i, _):
        rows_0[lax.div(i, 8), pl.ds(lax.rem(i, 8) * 16, 16)] = jnp.zeros(
            (16,), jnp.float32)
        return 0
    lax.fori_loop(0, CHUNK * (D // 16), _zero_rows, 0)
    base = sid * ROWS_PER_TILE
    nz = ROWS_PER_TILE // CHUNK
    for z in range(nz):
        pltpu.async_copy(rows_0, acc.at[pl.ds(base + z * CHUNK, CHUNK)],
                         sem_z)
    if with_counts:
        for z in range(nz):
            pltpu.async_copy(rows_0.at[:, pl.ds(0, CW)],
                             cnt.at[pl.ds(base + z * CHUNK, CHUNK)], sem_z)

        def _fill_ones(i, _):
            ones_buf[i, pl.ds(0, CW)] = jnp.ones((CW,), jnp.float32)
            return 0
        lax.fori_loop(0, CHUNK, _fill_ones, 0)
    for z in range(nz):
        pltpu.make_async_copy(rows_0, acc.at[pl.ds(base + z * CHUNK, CHUNK)],
                              sem_z).wait()
    if with_counts:
        for z in range(nz):
            pltpu.make_async_copy(rows_0.at[:, pl.ds(0, CW)],
                                  cnt.at[pl.ds(base + z * CHUNK, CHUNK)],
                                  sem_z).wait()
    plsc.subcore_barrier()

    QC = CHUNK // 4
    q_sls = [pl.ds(k * QC, QC) for k in range(4)]

    def _off(g):
        return (start + g) * CHUNK

    def _issue_idx(g, q, sync=False):
        if sync:
            pltpu.sync_copy(src.at[pl.ds(_off(g), CHUNK)], ibuf_s.at[q])
            pltpu.sync_copy(dst.at[pl.ds(_off(g), CHUNK)], ibuf_d.at[q])
        else:
            pltpu.async_copy(src.at[pl.ds(_off(g), CHUNK)], ibuf_s.at[q],
                             sem_is[q])
            pltpu.async_copy(dst.at[pl.ds(_off(g), CHUNK)], ibuf_d.at[q],
                             sem_id[q])

    def _wait_idx(g, q):
        pltpu.make_async_copy(src.at[pl.ds(_off(g), CHUNK)], ibuf_s.at[q],
                              sem_is[q]).wait()
        pltpu.make_async_copy(dst.at[pl.ds(_off(g), CHUNK)], ibuf_d.at[q],
                              sem_id[q]).wait()

    def _issue_gathers(p):
        for k in range(4):
            pltpu.async_copy(table.at[ibuf_s.at[p, q_sls[k]]],
                             rows[p].at[q_sls[k]], sem_g[p][k])

    def _wait_gathers(p):
        for k in range(4):
            pltpu.make_async_copy(table.at[ibuf_s.at[p, q_sls[k]]],
                                  rows[p].at[q_sls[k]], sem_g[p][k]).wait()

    def _issue_scatters(p):
        pltpu.async_copy(rows[p], acc.at[ibuf_d.at[p]], sem_m[p], add=True)
        if with_counts:
            pltpu.async_copy(ones_buf, cnt.at[ibuf_d.at[p]], sem_c[p],
                             add=True)

    def _wait_scatters(p):
        pltpu.make_async_copy(rows[p], acc.at[ibuf_d.at[p]], sem_m[p]).wait()
        if with_counts:
            pltpu.make_async_copy(ones_buf, cnt.at[ibuf_d.at[p]],
                                  sem_c[p]).wait()

    _issue_idx(0, 0, sync=True)
    _issue_gathers(0)

    def _step(g, _):
        def do(p):
            q = 1 - p

            @pl.when(g >= 1)
            def _():
                _wait_scatters(q)

            @pl.when(g + 1 < nreal)
            def _():
                _issue_idx(g + 1, q)
            _wait_gathers(p)

            @pl.when(g + 1 < nreal)
            def _():
                _wait_idx(g + 1, q)
                _issue_gathers(q)
            _issue_scatters(p)

        par = lax.rem(g, 2)
        for br in range(2):
            @pl.when(par == br)
            def _(br=br):
                do(br)
        return 0

    lax.fori_loop(0, nreal, _step, 0)
    for br in range(2):
        @pl.when(lax.rem(nreal - 1, 2) == br)
        def _(br=br):
            _wait_scatters(br)
    plsc.subcore_barrier()

    pltpu.sync_copy(acc.at[pl.ds(base, ROWS_PER_TILE)],
                    out.at[cid, pl.ds(base, ROWS_PER_TILE)])
    if with_counts:
        pltpu.sync_copy(cnt.at[pl.ds(base, ROWS_PER_TILE)],
                        out_cnt.at[cid, pl.ds(base, ROWS_PER_TILE),
                                   pl.ds(0, CW)])


def _sc_aggregate(table, src, dst, with_counts):
    mesh = plsc.VectorSubcoreMesh(core_axis_name="c", subcore_axis_name="s")
    return pl.kernel(
        functools.partial(_sc_aggregate_body, with_counts),
        out_type=(jax.ShapeDtypeStruct((NC, N_ACC, D), jnp.float32),
                  jax.ShapeDtypeStruct((NC, N_ACC, D), jnp.float32)),
        mesh=mesh,
        compiler_params=pltpu.CompilerParams(use_tc_tiling_on_sc=False),
        scratch_types=[
            pltpu.VMEM((2, CHUNK), jnp.int32),
            pltpu.VMEM((2, CHUNK), jnp.int32),
            pltpu.VMEM((CHUNK, D), jnp.float32),
            pltpu.VMEM((CHUNK, D), jnp.float32),
            pltpu.VMEM_SHARED((N_ACC, D), jnp.float32),
            pltpu.VMEM_SHARED((N_ACC, CW), jnp.float32),
            pltpu.VMEM((CHUNK, CW), jnp.float32),
        ] + [pltpu.SemaphoreType.DMA] * 17,
    )(table, src, dst)


def _dense_body(apply_relu, p_ref, c_ref, x_ref, wl_ref, wr_ref, b_ref,
                o_ref):
    s = p_ref[0] + p_ref[1]
    cnt = c_ref[0, :, 0:1] + c_ref[1, :, 0:1]
    mean = s / jnp.maximum(cnt, 1.0)
    y = (jnp.dot(mean, wl_ref[...], preferred_element_type=jnp.float32)
         + jnp.dot(x_ref[...], wr_ref[...], preferred_element_type=jnp.float32)
         + b_ref[...])
    if apply_relu:
        y = jnp.maximum(y, 0.0)
    o_ref[...] = y


def _dense(partials, counts, x, W_l, W_r, b, apply_relu):
    B = 2000
    return pl.pallas_call(
        functools.partial(_dense_body, apply_relu),
        grid=(N_NODES // B,),
        in_specs=[
            pl.BlockSpec((NC, B, D), lambda i: (0, i, 0)),
            pl.BlockSpec((NC, B, D), lambda i: (0, i, 0)),
            pl.BlockSpec((B, D), lambda i: (i, 0)),
            pl.BlockSpec((D, D), lambda i: (0, 0)),
            pl.BlockSpec((D, D), lambda i: (0, 0)),
            pl.BlockSpec((1, D), lambda i: (0, 0)),
        ],
        out_specs=pl.BlockSpec((B, D), lambda i: (i, 0)),
        out_shape=jax.ShapeDtypeStruct((N_NODES, D), jnp.float32),
    )(partials, counts, x, W_l, W_r, b)


def kernel(x, edge_index, W1_l, W1_r, b1, W2_l, W2_r, b2):
    src = edge_index[0].astype(jnp.int32)
    dst = edge_index[1].astype(jnp.int32)

    p1, c1 = _sc_aggregate(x, src, dst, with_counts=True)
    h = _dense(p1, c1, x, W1_l, W1_r, b1.reshape(1, D), apply_relu=True)
    p2, _ = _sc_aggregate(h, src, dst, with_counts=False)
    out = _dense(p2, c1, h, W2_l, W2_r, b2.reshape(1, D), apply_relu=False)
    return out


# speedup vs baseline: 1.0204x; 1.0204x over previous
import functools

import jax
import jax.numpy as jnp
from jax import lax
from jax.experimental import pallas as pl
from jax.experimental.pallas import tpu as pltpu
from jax.experimental.pallas import tpu_sc as plsc

N_NODES = 10000
D = 128
E = 320000
NC, NS = 2, 16
NW = NC * NS
CHUNK = 128
N_CHUNKS = E // CHUNK
BASE_CPW = N_CHUNKS // NW
EXTRA = N_CHUNKS % NW
N_ACC = 10240
ROWS_PER_TILE = N_ACC // NS
CW = 16


def _sc_aggregate_body(with_counts, table, src, dst, out, out_cnt,
                       ibuf_s, ibuf_d, rows_0, rows_1, acc, cnt, ones_buf,
                       sem_is0, sem_is1, sem_id0, sem_id1,
                       sem_g0a, sem_g0b, sem_g1a, sem_g1b,
                       sem_m0, sem_m1, sem_c0, sem_c1, sem_z):
    cid = lax.axis_index("c")
    sid = lax.axis_index("s")
    wid = sid * NC + cid
    nreal = BASE_CPW + jnp.where(wid < EXTRA, 1, 0)
    start = BASE_CPW * wid + jnp.minimum(wid, EXTRA)

    rows = (rows_0, rows_1)
    sem_is = (sem_is0, sem_is1)
    sem_id = (sem_id0, sem_id1)
    sem_g = ((sem_g0a, sem_g0b), (sem_g1a, sem_g1b))
    sem_m = (sem_m0, sem_m1)
    sem_c = (sem_c0, sem_c1)

    def _zero_rows(i, _):
        rows_0[lax.div(i, 8), pl.ds(lax.rem(i, 8) * 16, 16)] = jnp.zeros(
            (16,), jnp.float32)
        return 0
    lax.fori_loop(0, CHUNK * (D // 16), _zero_rows, 0)
    base = sid * ROWS_PER_TILE
    nz = ROWS_PER_TILE // CHUNK
    for z in range(nz):
        pltpu.async_copy(rows_0, acc.at[pl.ds(base + z * CHUNK, CHUNK)],
                         sem_z)
    if with_counts:
        for z in range(nz):
            pltpu.async_copy(rows_0.at[:, pl.ds(0, CW)],
                             cnt.at[pl.ds(base + z * CHUNK, CHUNK)], sem_z)

        def _fill_ones(i, _):
            ones_buf[i, pl.ds(0, CW)] = jnp.ones((CW,), jnp.float32)
            return 0
        lax.fori_loop(0, CHUNK, _fill_ones, 0)
    for z in range(nz):
        pltpu.make_async_copy(rows_0, acc.at[pl.ds(base + z * CHUNK, CHUNK)],
                              sem_z).wait()
    if with_counts:
        for z in range(nz):
            pltpu.make_async_copy(rows_0.at[:, pl.ds(0, CW)],
                                  cnt.at[pl.ds(base + z * CHUNK, CHUNK)],
                                  sem_z).wait()
    plsc.subcore_barrier()

    HC = CHUNK // 2
    ha_sl = pl.ds(0, HC)
    hb_sl = pl.ds(HC, HC)

    def _off(g):
        return (start + g) * CHUNK

    def _issue_idx(g, q, sync=False):
        if sync:
            pltpu.sync_copy(src.at[pl.ds(_off(g), CHUNK)], ibuf_s.at[q])
            pltpu.sync_copy(dst.at[pl.ds(_off(g), CHUNK)], ibuf_d.at[q])
        else:
            pltpu.async_copy(src.at[pl.ds(_off(g), CHUNK)], ibuf_s.at[q],
                             sem_is[q])
            pltpu.async_copy(dst.at[pl.ds(_off(g), CHUNK)], ibuf_d.at[q],
                             sem_id[q])

    def _wait_idx(g, q):
        pltpu.make_async_copy(src.at[pl.ds(_off(g), CHUNK)], ibuf_s.at[q],
                              sem_is[q]).wait()
        pltpu.make_async_copy(dst.at[pl.ds(_off(g), CHUNK)], ibuf_d.at[q],
                              sem_id[q]).wait()

    def _issue_gathers(p):
        pltpu.async_copy(table.at[ibuf_s.at[p, ha_sl]], rows[p].at[ha_sl],
                         sem_g[p][0])
        pltpu.async_copy(table.at[ibuf_s.at[p, hb_sl]], rows[p].at[hb_sl],
                         sem_g[p][1])

    def _wait_gathers(p):
        pltpu.make_async_copy(table.at[ibuf_s.at[p, ha_sl]],
                              rows[p].at[ha_sl], sem_g[p][0]).wait()
        pltpu.make_async_copy(table.at[ibuf_s.at[p, hb_sl]],
                              rows[p].at[hb_sl], sem_g[p][1]).wait()

    def _issue_scatters(p):
        pltpu.async_copy(rows[p], acc.at[ibuf_d.at[p]], sem_m[p], add=True)
        if with_counts:
            pltpu.async_copy(ones_buf, cnt.at[ibuf_d.at[p]], sem_c[p],
                             add=True)

    def _wait_scatters(p):
        pltpu.make_async_copy(rows[p], acc.at[ibuf_d.at[p]], sem_m[p]).wait()
        if with_counts:
            pltpu.make_async_copy(ones_buf, cnt.at[ibuf_d.at[p]],
                                  sem_c[p]).wait()

    _issue_idx(0, 0, sync=True)
    _issue_gathers(0)

    def _step(g, _):
        def do(p):
            q = 1 - p

            @pl.when(g >= 1)
            def _():
                _wait_scatters(q)

            @pl.when(g + 1 < nreal)
            def _():
                _issue_idx(g + 1, q)
            _wait_gathers(p)

            @pl.when(g + 1 < nreal)
            def _():
                _wait_idx(g + 1, q)
                _issue_gathers(q)
            _issue_scatters(p)

        par = lax.rem(g, 2)
        for br in range(2):
            @pl.when(par == br)
            def _(br=br):
                do(br)
        return 0

    lax.fori_loop(0, nreal, _step, 0)
    for br in range(2):
        @pl.when(lax.rem(nreal - 1, 2) == br)
        def _(br=br):
            _wait_scatters(br)
    plsc.subcore_barrier()

    pltpu.sync_copy(acc.at[pl.ds(base, ROWS_PER_TILE)],
                    out.at[cid, pl.ds(base, ROWS_PER_TILE)])
    if with_counts:
        pltpu.sync_copy(cnt.at[pl.ds(base, ROWS_PER_TILE)],
                        out_cnt.at[cid, pl.ds(base, ROWS_PER_TILE),
                                   pl.ds(0, CW)])


def _sc_aggregate(table, src, dst, with_counts):
    mesh = plsc.VectorSubcoreMesh(core_axis_name="c", subcore_axis_name="s")
    return pl.kernel(
        functools.partial(_sc_aggregate_body, with_counts),
        out_type=(jax.ShapeDtypeStruct((NC, N_ACC, D), jnp.float32),
                  jax.ShapeDtypeStruct((NC, N_ACC, D), jnp.float32)),
        mesh=mesh,
        compiler_params=pltpu.CompilerParams(use_tc_tiling_on_sc=False),
        scratch_types=[
            pltpu.VMEM((2, CHUNK), jnp.int32),
            pltpu.VMEM((2, CHUNK), jnp.int32),
            pltpu.VMEM((CHUNK, D), jnp.float32),
            pltpu.VMEM((CHUNK, D), jnp.float32),
            pltpu.VMEM_SHARED((N_ACC, D), jnp.float32),
            pltpu.VMEM_SHARED((N_ACC, CW), jnp.float32),
            pltpu.VMEM((CHUNK, CW), jnp.float32),
        ] + [pltpu.SemaphoreType.DMA] * 13,
    )(table, src, dst)


def _dense_body(apply_relu, p_ref, c_ref, x_ref, wl_ref, wr_ref, b_ref,
                o_ref):
    s = p_ref[0] + p_ref[1]
    cnt = c_ref[0, :, 0:1] + c_ref[1, :, 0:1]
    mean = s / jnp.maximum(cnt, 1.0)
    y = (jnp.dot(mean, wl_ref[...], preferred_element_type=jnp.float32)
         + jnp.dot(x_ref[...], wr_ref[...], preferred_element_type=jnp.float32)
         + b_ref[...])
    if apply_relu:
        y = jnp.maximum(y, 0.0)
    o_ref[...] = y


def _dense(partials, counts, x, W_l, W_r, b, apply_relu):
    B = 2000
    return pl.pallas_call(
        functools.partial(_dense_body, apply_relu),
        grid=(N_NODES // B,),
        in_specs=[
            pl.BlockSpec((NC, B, D), lambda i: (0, i, 0)),
            pl.BlockSpec((NC, B, D), lambda i: (0, i, 0)),
            pl.BlockSpec((B, D), lambda i: (i, 0)),
            pl.BlockSpec((D, D), lambda i: (0, 0)),
            pl.BlockSpec((D, D), lambda i: (0, 0)),
            pl.BlockSpec((1, D), lambda i: (0, 0)),
        ],
        out_specs=pl.BlockSpec((B, D), lambda i: (i, 0)),
        out_shape=jax.ShapeDtypeStruct((N_NODES, D), jnp.float32),
    )(partials, counts, x, W_l, W_r, b)


def kernel(x, edge_index, W1_l, W1_r, b1, W2_l, W2_r, b2):
    src = edge_index[0].astype(jnp.int32)
    dst = edge_index[1].astype(jnp.int32)

    p1, c1 = _sc_aggregate(x, src, dst, with_counts=True)
    h = _dense(p1, c1, x, W1_l, W1_r, b1.reshape(1, D), apply_relu=True)
    p2, _ = _sc_aggregate(h, src, dst, with_counts=False)
    out = _dense(p2, c1, h, W2_l, W2_r, b2.reshape(1, D), apply_relu=False)
    return out


# speedup vs baseline: 1.0594x; 1.0382x over previous
import functools

import jax
import jax.numpy as jnp
from jax import lax
from jax.experimental import pallas as pl
from jax.experimental.pallas import tpu as pltpu
from jax.experimental.pallas import tpu_sc as plsc

N_NODES = 10000
D = 128
E = 320000
NC, NS = 2, 16
NW = NC * NS
CHUNK = 128
N_CHUNKS = E // CHUNK
BASE_CPW = N_CHUNKS // NW
EXTRA = N_CHUNKS % NW
N_ACC = 10240
ROWS_PER_TILE = N_ACC // NS
CW = 16


def _sc_aggregate_body(with_counts, table, src, dst, out, out_cnt,
                       ibuf_s, ibuf_d, rows_0, rows_1, acc, cnt, ones_buf,
                       sem_is0, sem_is1, sem_id0, sem_id1,
                       sem_g0a, sem_g0b, sem_g1a, sem_g1b,
                       sem_m0, sem_m1, sem_c0, sem_c1, sem_z):
    cid = lax.axis_index("c")
    sid = lax.axis_index("s")
    wid = sid * NC + cid
    nreal = BASE_CPW + jnp.where(wid < EXTRA, 1, 0)
    start = BASE_CPW * wid + jnp.minimum(wid, EXTRA)

    rows = (rows_0, rows_1)
    sem_is = (sem_is0, sem_is1)
    sem_id = (sem_id0, sem_id1)
    sem_g = ((sem_g0a, sem_g0b), (sem_g1a, sem_g1b))
    sem_m = (sem_m0, sem_m1)
    sem_c = (sem_c0, sem_c1)

    def _zero_rows(i, _):
        rows_0[lax.div(i, 8), pl.ds(lax.rem(i, 8) * 16, 16)] = jnp.zeros(
            (16,), jnp.float32)
        return 0
    lax.fori_loop(0, CHUNK * (D // 16), _zero_rows, 0)
    base = sid * ROWS_PER_TILE
    nz = ROWS_PER_TILE // CHUNK
    for z in range(nz):
        pltpu.async_copy(rows_0, acc.at[pl.ds(base + z * CHUNK, CHUNK)],
                         sem_z)
    if with_counts:
        for z in range(nz):
            pltpu.async_copy(rows_0.at[:, pl.ds(0, CW)],
                             cnt.at[pl.ds(base + z * CHUNK, CHUNK)], sem_z)

        def _fill_ones(i, _):
            ones_buf[i, pl.ds(0, CW)] = jnp.ones((CW,), jnp.float32)
            return 0
        lax.fori_loop(0, CHUNK, _fill_ones, 0)
    for z in range(nz):
        pltpu.make_async_copy(rows_0, acc.at[pl.ds(base + z * CHUNK, CHUNK)],
                              sem_z).wait()
    if with_counts:
        for z in range(nz):
            pltpu.make_async_copy(rows_0.at[:, pl.ds(0, CW)],
                                  cnt.at[pl.ds(base + z * CHUNK, CHUNK)],
                                  sem_z).wait()
    plsc.subcore_barrier()

    HC = CHUNK // 2
    ha_sl = pl.ds(0, HC)
    hb_sl = pl.ds(HC, HC)

    def _off(g):
        return (start + g) * CHUNK

    def _issue_idx(g, q, sync=False):
        if sync:
            pltpu.sync_copy(src.at[pl.ds(_off(g), CHUNK)], ibuf_s.at[q])
            pltpu.sync_copy(dst.at[pl.ds(_off(g), CHUNK)], ibuf_d.at[q])
        else:
            pltpu.async_copy(src.at[pl.ds(_off(g), CHUNK)], ibuf_s.at[q],
                             sem_is[q])
            pltpu.async_copy(dst.at[pl.ds(_off(g), CHUNK)], ibuf_d.at[q],
                             sem_id[q])

    def _wait_idx(g, q):
        pltpu.make_async_copy(src.at[pl.ds(_off(g), CHUNK)], ibuf_s.at[q],
                              sem_is[q]).wait()
        pltpu.make_async_copy(dst.at[pl.ds(_off(g), CHUNK)], ibuf_d.at[q],
                              sem_id[q]).wait()

    def _issue_gathers(p):
        pltpu.async_copy(table.at[ibuf_s.at[p, ha_sl]], rows[p].at[ha_sl],
                         sem_g[p][0])
        pltpu.async_copy(table.at[ibuf_s.at[p, hb_sl]], rows[p].at[hb_sl],
                         sem_g[p][1])

    def _wait_gathers(p):
        pltpu.make_async_copy(table.at[ibuf_s.at[p, ha_sl]],
                              rows[p].at[ha_sl], sem_g[p][0]).wait()
        pltpu.make_async_copy(table.at[ibuf_s.at[p, hb_sl]],
                              rows[p].at[hb_sl], sem_g[p][1]).wait()

    def _issue_scatters(p):
        pltpu.async_copy(rows[p], acc.at[ibuf_d.at[p]], sem_m[p], add=True)
        if with_counts:
            pltpu.async_copy(ones_buf, cnt.at[ibuf_d.at[p]], sem_c[p],
                             add=True)

    def _wait_scatters(p):
        pltpu.make_async_copy(rows[p], acc.at[ibuf_d.at[p]], sem_m[p]).wait()
        if with_counts:
            pltpu.make_async_copy(ones_buf, cnt.at[ibuf_d.at[p]],
                                  sem_c[p]).wait()

    _issue_idx(0, 0, sync=True)
    _issue_gathers(0)

    def _step(g, _):
        def do(p):
            q = 1 - p

            @pl.when(g >= 1)
            def _():
                _wait_scatters(q)

            @pl.when(g + 1 < nreal)
            def _():
                _issue_idx(g + 1, q)
            _wait_gathers(p)

            @pl.when(g + 1 < nreal)
            def _():
                _wait_idx(g + 1, q)
                _issue_gathers(q)
            _issue_scatters(p)

        par = lax.rem(g, 2)
        for br in range(2):
            @pl.when(par == br)
            def _(br=br):
                do(br)
        return 0

    lax.fori_loop(0, nreal, _step, 0)
    for br in range(2):
        @pl.when(lax.rem(nreal - 1, 2) == br)
        def _(br=br):
            _wait_scatters(br)
    plsc.subcore_barrier()

    pltpu.sync_copy(acc.at[pl.ds(base, ROWS_PER_TILE)],
                    out.at[cid, pl.ds(base, ROWS_PER_TILE)])
    if with_counts:
        pltpu.sync_copy(cnt.at[pl.ds(base, ROWS_PER_TILE)],
                        out_cnt.at[cid, pl.ds(base, ROWS_PER_TILE),
                                   pl.ds(0, CW)])


def _sc_aggregate(table, src, dst, with_counts):
    mesh = plsc.VectorSubcoreMesh(core_axis_name="c", subcore_axis_name="s")
    return pl.kernel(
        functools.partial(_sc_aggregate_body, with_counts),
        out_type=(jax.ShapeDtypeStruct((NC, N_ACC, D), jnp.float32),
                  jax.ShapeDtypeStruct((NC, N_ACC, D), jnp.float32)),
        mesh=mesh,
        compiler_params=pltpu.CompilerParams(use_tc_tiling_on_sc=False),
        scratch_types=[
            pltpu.VMEM((2, CHUNK), jnp.int32),
            pltpu.VMEM((2, CHUNK), jnp.int32),
            pltpu.VMEM((CHUNK, D), jnp.float32),
            pltpu.VMEM((CHUNK, D), jnp.float32),
            pltpu.VMEM_SHARED((N_ACC, D), jnp.float32),
            pltpu.VMEM_SHARED((N_ACC, CW), jnp.float32),
            pltpu.VMEM((CHUNK, CW), jnp.float32),
        ] + [pltpu.SemaphoreType.DMA] * 13,
    )(table, src, dst)


def _split_body(e_ref, s_ref, d_ref):
    s_ref[...] = e_ref[0]
    d_ref[...] = e_ref[1]


def _split_edges(edge_index):
    return pl.pallas_call(
        _split_body,
        out_shape=[jax.ShapeDtypeStruct((E,), jnp.int32),
                   jax.ShapeDtypeStruct((E,), jnp.int32)],
    )(edge_index)


def _dense_body(apply_relu, p_ref, c_ref, x_ref, wl_ref, wr_ref, b_ref,
                o_ref):
    s = p_ref[0] + p_ref[1]
    cnt = c_ref[0, :, 0:1] + c_ref[1, :, 0:1]
    mean = s / jnp.maximum(cnt, 1.0)
    y = (jnp.dot(mean, wl_ref[...], preferred_element_type=jnp.float32)
         + jnp.dot(x_ref[...], wr_ref[...], preferred_element_type=jnp.float32)
         + b_ref[...])
    if apply_relu:
        y = jnp.maximum(y, 0.0)
    o_ref[...] = y


def _dense(partials, counts, x, W_l, W_r, b, apply_relu):
    B = 2000
    return pl.pallas_call(
        functools.partial(_dense_body, apply_relu),
        grid=(N_NODES // B,),
        in_specs=[
            pl.BlockSpec((NC, B, D), lambda i: (0, i, 0)),
            pl.BlockSpec((NC, B, D), lambda i: (0, i, 0)),
            pl.BlockSpec((B, D), lambda i: (i, 0)),
            pl.BlockSpec((D, D), lambda i: (0, 0)),
            pl.BlockSpec((D, D), lambda i: (0, 0)),
            pl.BlockSpec((1, D), lambda i: (0, 0)),
        ],
        out_specs=pl.BlockSpec((B, D), lambda i: (i, 0)),
        out_shape=jax.ShapeDtypeStruct((N_NODES, D), jnp.float32),
    )(partials, counts, x, W_l, W_r, b)


def kernel(x, edge_index, W1_l, W1_r, b1, W2_l, W2_r, b2):
    src, dst = _split_edges(edge_index.astype(jnp.int32))

    p1, c1 = _sc_aggregate(x, src, dst, with_counts=True)
    h = _dense(p1, c1, x, W1_l, W1_r, b1.reshape(1, D), apply_relu=True)
    p2, _ = _sc_aggregate(h, src, dst, with_counts=False)
    out = _dense(p2, c1, h, W2_l, W2_r, b2.reshape(1, D), apply_relu=False)
    return out
